# baseline (speedup 1.0000x reference)
import functools

import jax
import jax.numpy as jnp
from jax.experimental import pallas as pl
from jax.experimental.pallas import tpu as pltpu

_K_RETAIN_MAX = 8


def _pick_bm(n):
    for bm in (256, 250, 200, 128, 100, 80, 50, 40, 25, 16, 8):
        if n % bm == 0 and bm % 8 == 0 and bm * n * 4 <= 9_000_000:
            return bm
    return n


def _fused_kernel(nb, k, m, x_ref, w_ref, b_ref, adj_ref, out_ref, nd_s, hp_s, keep_s):
    s = pl.program_id(0)
    bm = out_ref.shape[0]
    t = s - nb

    @pl.when(s < nb)
    def _phase0():
        sl = pl.ds(s * bm, bm)
        nd_s[sl, :] = jnp.sum(adj_ref[:, :], axis=1, keepdims=True) + 1.0
        hp_s[sl, :] = jnp.dot(
            x_ref[:, :], w_ref[:, :], preferred_element_type=jnp.float32
        )

    if k > 0:
        @pl.when((s >= nb - 1 - k) & (s < nb - 1))
        def _retain():
            r = s - (nb - 1 - k)
            keep_s[r] = adj_ref[:, :].astype(jnp.bfloat16)

    def _emit(src, blk):
        acc = jnp.dot(src, hp_s[:, :], preferred_element_type=jnp.float32)
        sl = pl.ds(blk * bm, bm)
        out_ref[:, :] = nd_s[sl, :] * (acc + hp_s[sl, :]) + b_ref[:, :]

    @pl.when(s == nb - 1)
    def _finalize_and_emit_last():
        dis = jax.lax.rsqrt(nd_s[:, :])
        dis = jnp.where(jnp.isinf(dis), 0.0, dis)
        nd_s[:, :] = dis
        hp_s[:, :] = dis * hp_s[:, :]
        _emit(adj_ref[:, :], nb - 1)

    if k == 0:
        @pl.when(s >= nb)
        def _hbm_only():
            _emit(adj_ref[:, :], t)
    elif m == 1:
        @pl.when(s >= nb)
        def _retained_only():
            _emit(keep_s[t].astype(jnp.float32), nb - 1 - k + t)
    else:
        q = t // m
        retained = (t % m == m - 1) & (q < k)

        @pl.when((s >= nb) & retained)
        def _from_vmem():
            _emit(keep_s[q].astype(jnp.float32), nb - 1 - k + q)

        @pl.when((s >= nb) & jnp.logical_not(retained))
        def _from_hbm():
            _emit(adj_ref[:, :], t - jnp.minimum(q, k))


def kernel(x, edge_index, edge_weight, W, b):
    n, d_in = x.shape
    d_out = W.shape[1]
    bm = _pick_bm(n)
    nb = n // bm
    k = min(_K_RETAIN_MAX, nb - 1)
    m = max(1, (nb - 1) // k) if k > 0 else 1

    def adj_idx(s):
        if k == 0 or nb == 1:
            return (jnp.where(s < nb, s, s - nb), 0)
        t = s - nb
        if m == 1:
            p1 = nb - 1
        else:
            q = t // m
            retained = (t % m == m - 1) & (q < k)
            h = t - jnp.minimum(q, k)
            hprev = (t - 1) - jnp.minimum((t - 1) // m, k)
            p1 = jnp.where(retained, hprev, h)
        return (jnp.where(s < nb, s, p1), 0)

    def x_idx(s):
        return (jnp.minimum(s, nb - 1), 0)

    def out_idx(s):
        if k == 0 or nb == 1:
            return (jnp.where(s < nb, nb - 1, s - nb), 0)
        t = s - nb
        if m == 1:
            p1 = nb - 1 - k + t
        else:
            q = t // m
            retained = (t % m == m - 1) & (q < k)
            p1 = jnp.where(retained, nb - 1 - k + q, t - jnp.minimum(q, k))
        return (jnp.where(s < nb, nb - 1, p1), 0)

    out = pl.pallas_call(
        functools.partial(_fused_kernel, nb, k, m),
        grid=(2 * nb - 1,),
        in_specs=[
            pl.BlockSpec((bm, d_in), x_idx),
            pl.BlockSpec((d_in, d_out), lambda s: (0, 0)),
            pl.BlockSpec((1, d_out), lambda s: (0, 0)),
            pl.BlockSpec((bm, n), adj_idx),
        ],
        out_specs=pl.BlockSpec((bm, d_out), out_idx),
        out_shape=jax.ShapeDtypeStruct((n, d_out), jnp.float32),
        scratch_shapes=[
            pltpu.VMEM((n, 1), jnp.float32),
            pltpu.VMEM((n, d_out), jnp.float32),
            pltpu.VMEM((max(1, k), bm, n), jnp.bfloat16),
        ],
        compiler_params=pltpu.CompilerParams(
            dimension_semantics=("arbitrary",),
            vmem_limit_bytes=64 * 1024 * 1024,
        ),
    )(x, W, b.reshape(1, d_out), edge_index)

    return out


# speedup vs baseline: 1.0283x; 1.0283x over previous
import functools

import jax
import jax.numpy as jnp
from jax.experimental import pallas as pl
from jax.experimental.pallas import tpu as pltpu

_K_RETAIN_MAX = 2


def _pick_bm(n):
    for bm in (512, 500, 400, 256, 250, 200, 128, 100, 80, 50, 40, 25, 16, 8):
        if n % bm == 0 and bm % 8 == 0 and bm * n * 4 <= 17_000_000:
            return bm
    return n


def _fused_kernel(nb, k, m, x_ref, w_ref, b_ref, adj_ref, out_ref, nd_s, hp_s, keep_s):
    s = pl.program_id(0)
    bm = out_ref.shape[0]
    t = s - nb

    @pl.when(s < nb)
    def _phase0():
        sl = pl.ds(s * bm, bm)
        nd_s[sl, :] = jnp.sum(adj_ref[:, :], axis=1, keepdims=True) + 1.0
        hp_s[sl, :] = jnp.dot(
            x_ref[:, :], w_ref[:, :], preferred_element_type=jnp.float32
        )

    if k > 0:
        @pl.when((s >= nb - 1 - k) & (s < nb - 1))
        def _retain():
            r = s - (nb - 1 - k)
            keep_s[r] = adj_ref[:, :].astype(jnp.bfloat16)

    def _emit(src, blk):
        acc = jnp.dot(src, hp_s[:, :], preferred_element_type=jnp.float32)
        sl = pl.ds(blk * bm, bm)
        out_ref[:, :] = nd_s[sl, :] * (acc + hp_s[sl, :]) + b_ref[:, :]

    @pl.when(s == nb - 1)
    def _finalize_and_emit_last():
        dis = jax.lax.rsqrt(nd_s[:, :])
        dis = jnp.where(jnp.isinf(dis), 0.0, dis)
        nd_s[:, :] = dis
        hp_s[:, :] = dis * hp_s[:, :]
        _emit(adj_ref[:, :], nb - 1)

    if k == 0:
        @pl.when(s >= nb)
        def _hbm_only():
            _emit(adj_ref[:, :], t)
    elif m == 1:
        @pl.when(s >= nb)
        def _retained_only():
            _emit(keep_s[t].astype(jnp.float32), nb - 1 - k + t)
    else:
        q = t // m
        retained = (t % m == m - 1) & (q < k)

        @pl.when((s >= nb) & retained)
        def _from_vmem():
            _emit(keep_s[q].astype(jnp.float32), nb - 1 - k + q)

        @pl.when((s >= nb) & jnp.logical_not(retained))
        def _from_hbm():
            _emit(adj_ref[:, :], t - jnp.minimum(q, k))


def kernel(x, edge_index, edge_weight, W, b):
    n, d_in = x.shape
    d_out = W.shape[1]
    bm = _pick_bm(n)
    nb = n // bm
    k = min(_K_RETAIN_MAX, nb - 1)
    m = max(1, (nb - 1) // k) if k > 0 else 1

    def adj_idx(s):
        if k == 0 or nb == 1:
            return (jnp.where(s < nb, s, s - nb), 0)
        t = s - nb
        if m == 1:
            p1 = nb - 1
        else:
            q = t // m
            retained = (t % m == m - 1) & (q < k)
            h = t - jnp.minimum(q, k)
            hprev = (t - 1) - jnp.minimum((t - 1) // m, k)
            p1 = jnp.where(retained, hprev, h)
        return (jnp.where(s < nb, s, p1), 0)

    def x_idx(s):
        return (jnp.minimum(s, nb - 1), 0)

    def out_idx(s):
        if k == 0 or nb == 1:
            return (jnp.where(s < nb, nb - 1, s - nb), 0)
        t = s - nb
        if m == 1:
            p1 = nb - 1 - k + t
        else:
            q = t // m
            retained = (t % m == m - 1) & (q < k)
            p1 = jnp.where(retained, nb - 1 - k + q, t - jnp.minimum(q, k))
        return (jnp.where(s < nb, nb - 1, p1), 0)

    out = pl.pallas_call(
        functools.partial(_fused_kernel, nb, k, m),
        grid=(2 * nb - 1,),
        in_specs=[
            pl.BlockSpec((bm, d_in), x_idx),
            pl.BlockSpec((d_in, d_out), lambda s: (0, 0)),
            pl.BlockSpec((1, d_out), lambda s: (0, 0)),
            pl.BlockSpec((bm, n), adj_idx),
        ],
        out_specs=pl.BlockSpec((bm, d_out), out_idx),
        out_shape=jax.ShapeDtypeStruct((n, d_out), jnp.float32),
        scratch_shapes=[
            pltpu.VMEM((n, 1), jnp.float32),
            pltpu.VMEM((n, d_out), jnp.float32),
            pltpu.VMEM((max(1, k), bm, n), jnp.bfloat16),
        ],
        compiler_params=pltpu.CompilerParams(
            dimension_semantics=("arbitrary",),
            vmem_limit_bytes=64 * 1024 * 1024,
        ),
    )(x, W, b.reshape(1, d_out), edge_index)

    return out
